# Initial kernel scaffold; baseline (speedup 1.0000x reference)
import jax
import jax.numpy as jnp
from jax.experimental import pallas as pl


def kernel(flat, row_lengths):
    raise NotImplementedError("write your pallas kernel here")


# speedup vs baseline: 243.8354x; 243.8354x over previous
import functools

import jax
import jax.numpy as jnp
from jax import lax
from jax.experimental import pallas as pl
from jax.experimental.pallas import tpu as pltpu
from jax.experimental.pallas import tpu_sc as plsc

_NUM_CORES = 2
_NUM_SUBCORES = 16
_NUM_WORKERS = _NUM_CORES * _NUM_SUBCORES


def kernel(flat, row_lengths):
    total = flat.shape[0]
    chunk = -(-total // _NUM_WORKERS)
    chunk = ((chunk + 7) // 8) * 8
    padded = chunk * _NUM_WORKERS
    flat_p = jnp.pad(flat, (0, padded - total))

    mesh = plsc.VectorSubcoreMesh(core_axis_name="c", subcore_axis_name="s")

    @functools.partial(
        pl.kernel,
        mesh=mesh,
        out_type=jax.ShapeDtypeStruct((padded,), jnp.float32),
        scratch_types=[pltpu.VMEM((chunk,), jnp.float32)],
    )
    def ragged_roundtrip(in_hbm, out_hbm, buf):
        wid = lax.axis_index("s") * _NUM_CORES + lax.axis_index("c")
        base = wid * chunk
        pltpu.sync_copy(in_hbm.at[pl.ds(base, chunk)], buf)
        pltpu.sync_copy(buf, out_hbm.at[pl.ds(base, chunk)])

    out = ragged_roundtrip(flat_p)
    return out[:total], row_lengths
